# baseline (speedup 1.0000x reference)
import jax
import jax.numpy as jnp
from jax import lax
from jax.experimental import pallas as pl
from jax.experimental.pallas import tpu as pltpu
from jax.experimental.pallas import tpu_sc as plsc

B, C, H, W = 4, 96, 96 * 4, 96 * 4
HW = H * W
NC, NS, L = 2, 16, 16
NW = NC * NS
ROWS_PER_W = (B * H) // NW
CH = 64
NCHUNK = W // CH
NQ = ROWS_PER_W * NCHUNK
NG = CH // L
P1 = 128
NP1 = (2 * HW // NS) // P1
P1STRIPE = 2 * HW // NS // 2


def _sc_warp_kernel(vol_hbm, trf_hbm, out_hbm, table_hbm,
                    sbuf, tbuf, flow_v, ibuf, wbuf, vbuf, obuf,
                    psem, wsem, gsem, osem, fsem):
  core = lax.axis_index("c")
  sid = lax.axis_index("s")
  b = 2 * core + sid // (NS // 2)
  i_base = lax.rem(sid, NS // 2) * ROWS_PER_W
  tb = b * HW

  iota = lax.iota(jnp.int32, L)
  iota_f = iota.astype(jnp.float32)

  p1b = b * HW + lax.rem(sid, NS // 2) * P1STRIPE

  def p1_src(s_i, buf):
    off = pl.multiple_of(p1b - tb + s_i * P1, P1)
    return vol_hbm.at[b, :, pl.ds(off, P1)], sbuf.at[buf]

  def p1_fire_in(s_i, buf):
    src, dst = p1_src(s_i, buf)
    pltpu.async_copy(src, dst, psem)

  def p1_wait_in(s_i, buf):
    src, dst = p1_src(s_i, buf)
    pltpu.make_async_copy(src, dst, psem).wait()

  def p1_dst(s_i, buf):
    off = pl.multiple_of(p1b + s_i * P1, P1)
    return tbuf.at[buf], table_hbm.at[pl.ds(off, P1)]

  def p1_fire_out(s_i, buf):
    src, dst = p1_dst(s_i, buf)
    pltpu.async_copy(src, dst, wsem)

  def p1_wait_out(s_i, buf):
    src, dst = p1_dst(s_i, buf)
    pltpu.make_async_copy(src, dst, wsem).wait()

  def p1_transpose(buf):
    def px_body(p, _):
      pv = jnp.full((L,), p, jnp.int32)
      for cg in range(C // L):
        v = plsc.load_gather(sbuf.at[buf], [cg * L + iota, pv])
        tbuf[buf, p, pl.ds(cg * L, L)] = v
      return 0

    lax.fori_loop(0, P1, px_body, 0, unroll=2)

  p1_fire_in(jnp.int32(0), jnp.int32(0))

  def p1_body(s_i, _):
    buf = lax.rem(s_i, 2)
    nbuf = 1 - buf

    @pl.when(s_i + 1 < NP1)
    def _fire_next():
      p1_fire_in(s_i + 1, nbuf)

    p1_wait_in(s_i, buf)
    p1_transpose(buf)

    @pl.when(s_i >= 1)
    def _drain_prev():
      p1_wait_out(s_i - 1, nbuf)

    p1_fire_out(s_i, buf)
    return 0

  lax.fori_loop(0, NP1, p1_body, 0)
  p1_wait_out(jnp.int32(NP1 - 1), jnp.int32((NP1 - 1) % 2))

  plsc.subcore_barrier()

  def flow_off(k, z):
    return pl.multiple_of((b * 2 * H + z * H + (i_base + k)) * W, W)

  def fire_flow_prefetch(k):
    par = lax.rem(k, 2)
    pltpu.async_copy(trf_hbm.at[pl.ds(flow_off(k, 0), W)],
                     flow_v.at[par, 0], fsem)
    pltpu.async_copy(trf_hbm.at[pl.ds(flow_off(k, 1), W)],
                     flow_v.at[par, 1], fsem)

  def wait_flow_prefetch(k):
    par = lax.rem(k, 2)
    pltpu.make_async_copy(trf_hbm.at[pl.ds(flow_off(k, 0), W)],
                          flow_v.at[par, 0], fsem).wait()
    pltpu.make_async_copy(trf_hbm.at[pl.ds(flow_off(k, 1), W)],
                          flow_v.at[par, 1], fsem).wait()

  def compute_chunk(k, t, bufi):
    par = lax.rem(k, 2)
    i_f = (i_base + k).astype(jnp.float32)
    for g in range(NG):
      sl = pl.ds(g * L, L)
      jpos = t * CH + g * L
      fi = flow_v[par, 0, pl.ds(jpos, L)]
      fj = flow_v[par, 1, pl.ds(jpos, L)]
      loc_i = jnp.clip(i_f + fi, 0.0, float(H - 1))
      loc_j = jnp.clip(jpos.astype(jnp.float32) + iota_f + fj,
                       0.0, float(W - 1))
      i0 = jnp.minimum(loc_i.astype(jnp.int32), H - 2)
      j0 = jnp.minimum(loc_j.astype(jnp.int32), W - 2)
      wi = loc_i - i0.astype(jnp.float32)
      wj = loc_j - j0.astype(jnp.float32)
      base_idx = tb + i0 * W + j0
      ibuf[bufi, 0, sl] = base_idx
      ibuf[bufi, 1, sl] = base_idx + 1
      ibuf[bufi, 2, sl] = base_idx + W
      ibuf[bufi, 3, sl] = base_idx + (W + 1)
      wbuf[bufi, 0, sl] = (1.0 - wi) * (1.0 - wj)
      wbuf[bufi, 1, sl] = (1.0 - wi) * wj
      wbuf[bufi, 2, sl] = wi * (1.0 - wj)
      wbuf[bufi, 3, sl] = wi * wj

  def fire_gathers(bufi):
    for n in range(4):
      pltpu.async_copy(table_hbm.at[ibuf.at[bufi, n]], vbuf.at[bufi, n], gsem)

  def wait_gathers(bufi):
    for n in range(4):
      pltpu.make_async_copy(table_hbm.at[ibuf.at[bufi, n]],
                            vbuf.at[bufi, n], gsem).wait()

  def combine(bufi):
    def group_body(g, _):
      base = g * L
      w00v = wbuf[bufi, 0, pl.ds(base, L)]
      w01v = wbuf[bufi, 1, pl.ds(base, L)]
      w10v = wbuf[bufi, 2, pl.ds(base, L)]
      w11v = wbuf[bufi, 3, pl.ds(base, L)]

      def lane_body(l, _):
        lsplat = jnp.full((L,), l, jnp.int32)
        w00 = jnp.take_along_axis(w00v, lsplat, axis=0,
                                  mode="promise_in_bounds")
        w01 = jnp.take_along_axis(w01v, lsplat, axis=0,
                                  mode="promise_in_bounds")
        w10 = jnp.take_along_axis(w10v, lsplat, axis=0,
                                  mode="promise_in_bounds")
        w11 = jnp.take_along_axis(w11v, lsplat, axis=0,
                                  mode="promise_in_bounds")
        p = base + l
        for cg in range(C // L):
          sl = pl.ds(cg * L, L)
          obuf[bufi, p, sl] = (w00 * vbuf[bufi, 0, p, sl]
                               + w01 * vbuf[bufi, 1, p, sl]
                               + w10 * vbuf[bufi, 2, p, sl]
                               + w11 * vbuf[bufi, 3, p, sl])
        return 0

      lax.fori_loop(0, L, lane_body, 0, unroll=4)
      return 0

    lax.fori_loop(0, NG, group_body, 0)

  def out_slice(k, t):
    gbase = pl.multiple_of(tb + (i_base + k) * W + t * CH, CH)
    return out_hbm.at[pl.ds(gbase, CH)]

  def fire_write(k, t, bufi):
    pltpu.async_copy(obuf.at[bufi], out_slice(k, t), osem)

  def wait_write(k, t, bufi):
    pltpu.make_async_copy(obuf.at[bufi], out_slice(k, t), osem).wait()

  zero = jnp.int32(0)
  pltpu.sync_copy(trf_hbm.at[pl.ds(flow_off(zero, 0), W)], flow_v.at[0, 0])
  pltpu.sync_copy(trf_hbm.at[pl.ds(flow_off(zero, 1), W)], flow_v.at[0, 1])
  compute_chunk(zero, zero, zero)
  fire_gathers(zero)
  fire_flow_prefetch(jnp.int32(1))

  def q_body(q, _):
    buf = lax.rem(q, 2)
    nbuf = 1 - buf
    nq = q + 1
    nk = nq // NCHUNK
    nt = lax.rem(nq, NCHUNK)
    k = q // NCHUNK
    t = lax.rem(q, NCHUNK)

    @pl.when(nq < NQ)
    def _stage_next():
      @pl.when(nt == 0)
      def _flow_ready():
        wait_flow_prefetch(nk)

      compute_chunk(nk, nt, nbuf)
      fire_gathers(nbuf)

      @pl.when(jnp.logical_and(nt == 0, nk + 1 < ROWS_PER_W))
      def _flow_next():
        fire_flow_prefetch(nk + 1)

    wait_gathers(buf)
    combine(buf)

    @pl.when(q >= 1)
    def _drain_prev_write():
      wait_write((q - 1) // NCHUNK, lax.rem(q - 1, NCHUNK), nbuf)

    fire_write(k, t, buf)
    return 0

  lax.fori_loop(0, NQ, q_body, 0)
  wait_write(jnp.int32(ROWS_PER_W - 1), jnp.int32(NCHUNK - 1),
             jnp.int32((NQ - 1) % 2))


@jax.jit
def kernel(vol, trf):
  vol3 = vol.reshape(B, C, HW)
  trf_flat = trf.reshape(B * 2 * H * W)

  mesh = plsc.VectorSubcoreMesh(core_axis_name="c", subcore_axis_name="s",
                                num_cores=NC, num_subcores=NS)
  f = pl.kernel(
      _sc_warp_kernel,
      out_type=(jax.ShapeDtypeStruct((B * HW, C), jnp.float32),
                jax.ShapeDtypeStruct((B * HW, C), jnp.float32)),
      mesh=mesh,
      scratch_types=[
          pltpu.VMEM((2, C, P1), jnp.float32),
          pltpu.VMEM((2, P1, C), jnp.float32),
          pltpu.VMEM((2, 2, W), jnp.float32),
          pltpu.VMEM((2, 4, CH), jnp.int32),
          pltpu.VMEM((2, 4, CH), jnp.float32),
          pltpu.VMEM((2, 4, CH, C), jnp.float32),
          pltpu.VMEM((2, CH, C), jnp.float32),
          pltpu.SemaphoreType.DMA,
          pltpu.SemaphoreType.DMA,
          pltpu.SemaphoreType.DMA,
          pltpu.SemaphoreType.DMA,
          pltpu.SemaphoreType.DMA,
      ],
      compiler_params=pltpu.CompilerParams(use_tc_tiling_on_sc=False,
                                           needs_layout_passes=False),
  )
  out, _ = f(vol3, trf_flat)
  return out.reshape(B, H, W, C)


# speedup vs baseline: 1.0007x; 1.0007x over previous
import jax
import jax.numpy as jnp
from jax import lax
from jax.experimental import pallas as pl
from jax.experimental.pallas import tpu as pltpu
from jax.experimental.pallas import tpu_sc as plsc

B, C, H, W = 4, 96, 96 * 4, 96 * 4
HW = H * W
NC, NS, L = 2, 16, 16
NW = NC * NS
ROWS_PER_W = (B * H) // NW
CH = 64
NCHUNK = W // CH
NQ = ROWS_PER_W * NCHUNK
NG = CH // L
P1 = 128
P1STRIPE = HW // (NS // 2)
NP1 = P1STRIPE // P1


def _sc_warp_kernel(vol_hbm, trf_hbm, out_hbm, table_hbm,
                    sbuf, tbuf, flow_v, ibuf, wbuf, vbuf, obuf,
                    psem, wsem, gsem, osem, fsem):
  core = lax.axis_index("c")
  sid = lax.axis_index("s")
  b = 2 * core + sid // (NS // 2)
  i_base = lax.rem(sid, NS // 2) * ROWS_PER_W
  tb = b * HW

  iota = lax.iota(jnp.int32, L)
  iota_f = iota.astype(jnp.float32)

  p1b = b * HW + lax.rem(sid, NS // 2) * P1STRIPE

  def p1_src(s_i, buf):
    off = pl.multiple_of(p1b - tb + s_i * P1, P1)
    return vol_hbm.at[b, :, pl.ds(off, P1)], sbuf.at[buf]

  def p1_fire_in(s_i, buf):
    src, dst = p1_src(s_i, buf)
    pltpu.async_copy(src, dst, psem)

  def p1_wait_in(s_i, buf):
    src, dst = p1_src(s_i, buf)
    pltpu.make_async_copy(src, dst, psem).wait()

  def p1_dst(s_i, buf):
    off = pl.multiple_of(p1b + s_i * P1, P1)
    return tbuf.at[buf], table_hbm.at[pl.ds(off, P1)]

  def p1_fire_out(s_i, buf):
    src, dst = p1_dst(s_i, buf)
    pltpu.async_copy(src, dst, wsem)

  def p1_wait_out(s_i, buf):
    src, dst = p1_dst(s_i, buf)
    pltpu.make_async_copy(src, dst, wsem).wait()

  def p1_transpose(buf):
    def px_body(p, _):
      pv = jnp.full((L,), p, jnp.int32)
      for cg in range(C // L):
        v = plsc.load_gather(sbuf.at[buf], [cg * L + iota, pv])
        tbuf[buf, p, pl.ds(cg * L, L)] = v
      return 0

    lax.fori_loop(0, P1, px_body, 0, unroll=2)

  p1_fire_in(jnp.int32(0), jnp.int32(0))

  def p1_body(s_i, _):
    buf = lax.rem(s_i, 2)
    nbuf = 1 - buf

    @pl.when(s_i + 1 < NP1)
    def _fire_next():
      p1_fire_in(s_i + 1, nbuf)

    p1_wait_in(s_i, buf)
    p1_transpose(buf)

    @pl.when(s_i >= 1)
    def _drain_prev():
      p1_wait_out(s_i - 1, nbuf)

    p1_fire_out(s_i, buf)
    return 0

  lax.fori_loop(0, NP1, p1_body, 0)
  p1_wait_out(jnp.int32(NP1 - 1), jnp.int32((NP1 - 1) % 2))

  plsc.subcore_barrier()

  def flow_off(k, z):
    return pl.multiple_of((b * 2 * H + z * H + (i_base + k)) * W, W)

  def fire_flow_prefetch(k):
    par = lax.rem(k, 2)
    pltpu.async_copy(trf_hbm.at[pl.ds(flow_off(k, 0), W)],
                     flow_v.at[par, 0], fsem)
    pltpu.async_copy(trf_hbm.at[pl.ds(flow_off(k, 1), W)],
                     flow_v.at[par, 1], fsem)

  def wait_flow_prefetch(k):
    par = lax.rem(k, 2)
    pltpu.make_async_copy(trf_hbm.at[pl.ds(flow_off(k, 0), W)],
                          flow_v.at[par, 0], fsem).wait()
    pltpu.make_async_copy(trf_hbm.at[pl.ds(flow_off(k, 1), W)],
                          flow_v.at[par, 1], fsem).wait()

  def compute_chunk(k, t, bufi):
    par = lax.rem(k, 2)
    i_f = (i_base + k).astype(jnp.float32)
    for g in range(NG):
      sl = pl.ds(g * L, L)
      jpos = t * CH + g * L
      fi = flow_v[par, 0, pl.ds(jpos, L)]
      fj = flow_v[par, 1, pl.ds(jpos, L)]
      loc_i = jnp.clip(i_f + fi, 0.0, float(H - 1))
      loc_j = jnp.clip(jpos.astype(jnp.float32) + iota_f + fj,
                       0.0, float(W - 1))
      i0 = jnp.minimum(loc_i.astype(jnp.int32), H - 2)
      j0 = jnp.minimum(loc_j.astype(jnp.int32), W - 2)
      wi = loc_i - i0.astype(jnp.float32)
      wj = loc_j - j0.astype(jnp.float32)
      base_idx = tb + i0 * W + j0
      ibuf[bufi, 0, sl] = base_idx
      ibuf[bufi, 1, sl] = base_idx + 1
      ibuf[bufi, 2, sl] = base_idx + W
      ibuf[bufi, 3, sl] = base_idx + (W + 1)
      wbuf[bufi, 0, sl] = (1.0 - wi) * (1.0 - wj)
      wbuf[bufi, 1, sl] = (1.0 - wi) * wj
      wbuf[bufi, 2, sl] = wi * (1.0 - wj)
      wbuf[bufi, 3, sl] = wi * wj

  def fire_gathers(bufi):
    for n in range(4):
      pltpu.async_copy(table_hbm.at[ibuf.at[bufi, n]], vbuf.at[bufi, n], gsem)

  def wait_gathers(bufi):
    for n in range(4):
      pltpu.make_async_copy(table_hbm.at[ibuf.at[bufi, n]],
                            vbuf.at[bufi, n], gsem).wait()

  def combine(bufi):
    def group_body(g, _):
      base = g * L
      w00v = wbuf[bufi, 0, pl.ds(base, L)]
      w01v = wbuf[bufi, 1, pl.ds(base, L)]
      w10v = wbuf[bufi, 2, pl.ds(base, L)]
      w11v = wbuf[bufi, 3, pl.ds(base, L)]

      def lane_body(l, _):
        lsplat = jnp.full((L,), l, jnp.int32)
        w00 = jnp.take_along_axis(w00v, lsplat, axis=0,
                                  mode="promise_in_bounds")
        w01 = jnp.take_along_axis(w01v, lsplat, axis=0,
                                  mode="promise_in_bounds")
        w10 = jnp.take_along_axis(w10v, lsplat, axis=0,
                                  mode="promise_in_bounds")
        w11 = jnp.take_along_axis(w11v, lsplat, axis=0,
                                  mode="promise_in_bounds")
        p = base + l
        for cg in range(C // L):
          sl = pl.ds(cg * L, L)
          obuf[bufi, p, sl] = (w00 * vbuf[bufi, 0, p, sl]
                               + w01 * vbuf[bufi, 1, p, sl]
                               + w10 * vbuf[bufi, 2, p, sl]
                               + w11 * vbuf[bufi, 3, p, sl])
        return 0

      lax.fori_loop(0, L, lane_body, 0, unroll=4)
      return 0

    lax.fori_loop(0, NG, group_body, 0)

  def out_slice(k, t):
    gbase = pl.multiple_of(tb + (i_base + k) * W + t * CH, CH)
    return out_hbm.at[pl.ds(gbase, CH)]

  def fire_write(k, t, bufi):
    pltpu.async_copy(obuf.at[bufi], out_slice(k, t), osem)

  def wait_write(k, t, bufi):
    pltpu.make_async_copy(obuf.at[bufi], out_slice(k, t), osem).wait()

  zero = jnp.int32(0)
  pltpu.sync_copy(trf_hbm.at[pl.ds(flow_off(zero, 0), W)], flow_v.at[0, 0])
  pltpu.sync_copy(trf_hbm.at[pl.ds(flow_off(zero, 1), W)], flow_v.at[0, 1])
  compute_chunk(zero, zero, zero)
  fire_gathers(zero)
  fire_flow_prefetch(jnp.int32(1))

  def q_body(q, _):
    buf = lax.rem(q, 2)
    nbuf = 1 - buf
    nq = q + 1
    nk = nq // NCHUNK
    nt = lax.rem(nq, NCHUNK)
    k = q // NCHUNK
    t = lax.rem(q, NCHUNK)

    @pl.when(nq < NQ)
    def _stage_next():
      @pl.when(nt == 0)
      def _flow_ready():
        wait_flow_prefetch(nk)

      compute_chunk(nk, nt, nbuf)
      fire_gathers(nbuf)

      @pl.when(jnp.logical_and(nt == 0, nk + 1 < ROWS_PER_W))
      def _flow_next():
        fire_flow_prefetch(nk + 1)

    wait_gathers(buf)
    combine(buf)

    @pl.when(q >= 1)
    def _drain_prev_write():
      wait_write((q - 1) // NCHUNK, lax.rem(q - 1, NCHUNK), nbuf)

    fire_write(k, t, buf)
    return 0

  lax.fori_loop(0, NQ, q_body, 0)
  wait_write(jnp.int32(ROWS_PER_W - 1), jnp.int32(NCHUNK - 1),
             jnp.int32((NQ - 1) % 2))


def _build_kernel():
  mesh = plsc.VectorSubcoreMesh(core_axis_name="c", subcore_axis_name="s",
                                num_cores=NC, num_subcores=NS)
  return pl.kernel(
      _sc_warp_kernel,
      out_type=(jax.ShapeDtypeStruct((B * HW, C), jnp.float32),
                jax.ShapeDtypeStruct((B * HW, C), jnp.float32)),
      mesh=mesh,
      scratch_types=[
          pltpu.VMEM((2, C, P1), jnp.float32),
          pltpu.VMEM((2, P1, C), jnp.float32),
          pltpu.VMEM((2, 2, W), jnp.float32),
          pltpu.VMEM((2, 4, CH), jnp.int32),
          pltpu.VMEM((2, 4, CH), jnp.float32),
          pltpu.VMEM((2, 4, CH, C), jnp.float32),
          pltpu.VMEM((2, CH, C), jnp.float32),
          pltpu.SemaphoreType.DMA,
          pltpu.SemaphoreType.DMA,
          pltpu.SemaphoreType.DMA,
          pltpu.SemaphoreType.DMA,
          pltpu.SemaphoreType.DMA,
      ],
      compiler_params=pltpu.CompilerParams(use_tc_tiling_on_sc=False,
                                           needs_layout_passes=False),
  )


def kernel_debug(vol, trf):
  vol3 = vol.reshape(B, C, HW)
  trf_flat = trf.reshape(B * 2 * H * W)
  out, table = _build_kernel()(vol3, trf_flat)
  return out.reshape(B, H, W, C), table


@jax.jit
def kernel(vol, trf):
  vol3 = vol.reshape(B, C, HW)
  trf_flat = trf.reshape(B * 2 * H * W)
  out, _ = _build_kernel()(vol3, trf_flat)
  return out.reshape(B, H, W, C)


# speedup vs baseline: 1.5157x; 1.5147x over previous
import jax
import jax.numpy as jnp
from jax import lax
from jax.experimental import pallas as pl
from jax.experimental.pallas import tpu as pltpu
from jax.experimental.pallas import tpu_sc as plsc

B, C, H, W = 4, 96, 96 * 4, 96 * 4
HW = H * W
NC, NS, L = 2, 16, 16
NW = NC * NS
ROWS_PER_W = (B * H) // NW
CH = 64
NCHUNK = W // CH
NQ = ROWS_PER_W * NCHUNK
NG = CH // L
NB = 3


def _sc_warp_kernel(table_hbm, trf_hbm, out_hbm,
                    flow_v, ibuf, wbuf, vbuf, obuf, gsem, osem, fsem):
  wid = lax.axis_index("s") * NC + lax.axis_index("c")
  b = wid // (NW // B)
  i_base = lax.rem(wid, NW // B) * ROWS_PER_W
  tb = b * HW

  iota = lax.iota(jnp.int32, L)
  iota_f = iota.astype(jnp.float32)

  def flow_off(k, z):
    return pl.multiple_of((b * 2 * H + z * H + (i_base + k)) * W, W)

  def fire_flow_prefetch(k):
    par = lax.rem(k, 2)
    pltpu.async_copy(trf_hbm.at[pl.ds(flow_off(k, 0), W)],
                     flow_v.at[par, 0], fsem)
    pltpu.async_copy(trf_hbm.at[pl.ds(flow_off(k, 1), W)],
                     flow_v.at[par, 1], fsem)

  def wait_flow_prefetch(k):
    par = lax.rem(k, 2)
    pltpu.make_async_copy(trf_hbm.at[pl.ds(flow_off(k, 0), W)],
                          flow_v.at[par, 0], fsem).wait()
    pltpu.make_async_copy(trf_hbm.at[pl.ds(flow_off(k, 1), W)],
                          flow_v.at[par, 1], fsem).wait()

  def compute_chunk(k, t, bufi):
    par = lax.rem(k, 2)
    i_f = (i_base + k).astype(jnp.float32)
    for g in range(NG):
      sl = pl.ds(g * L, L)
      jpos = t * CH + g * L
      fi = flow_v[par, 0, pl.ds(jpos, L)]
      fj = flow_v[par, 1, pl.ds(jpos, L)]
      loc_i = jnp.clip(i_f + fi, 0.0, float(H - 1))
      loc_j = jnp.clip(jpos.astype(jnp.float32) + iota_f + fj,
                       0.0, float(W - 1))
      i0 = jnp.minimum(loc_i.astype(jnp.int32), H - 2)
      j0 = jnp.minimum(loc_j.astype(jnp.int32), W - 2)
      wi = loc_i - i0.astype(jnp.float32)
      wj = loc_j - j0.astype(jnp.float32)
      base_idx = tb + i0 * W + j0
      ibuf[bufi, 0, sl] = base_idx
      ibuf[bufi, 1, sl] = base_idx + 1
      ibuf[bufi, 2, sl] = base_idx + W
      ibuf[bufi, 3, sl] = base_idx + (W + 1)
      wbuf[bufi, 0, sl] = (1.0 - wi) * (1.0 - wj)
      wbuf[bufi, 1, sl] = (1.0 - wi) * wj
      wbuf[bufi, 2, sl] = wi * (1.0 - wj)
      wbuf[bufi, 3, sl] = wi * wj

  def fire_gathers(bufi):
    for n in range(4):
      pltpu.async_copy(table_hbm.at[ibuf.at[bufi, n]], vbuf.at[bufi, n], gsem)

  def wait_gathers(bufi):
    for n in range(4):
      pltpu.make_async_copy(table_hbm.at[ibuf.at[bufi, n]],
                            vbuf.at[bufi, n], gsem).wait()

  def combine(bufi, obi):
    def group_body(g, _):
      base = g * L
      w00v = wbuf[bufi, 0, pl.ds(base, L)]
      w01v = wbuf[bufi, 1, pl.ds(base, L)]
      w10v = wbuf[bufi, 2, pl.ds(base, L)]
      w11v = wbuf[bufi, 3, pl.ds(base, L)]

      def lane_body(l, _):
        lsplat = jnp.full((L,), l, jnp.int32)
        w00 = jnp.take_along_axis(w00v, lsplat, axis=0,
                                  mode="promise_in_bounds")
        w01 = jnp.take_along_axis(w01v, lsplat, axis=0,
                                  mode="promise_in_bounds")
        w10 = jnp.take_along_axis(w10v, lsplat, axis=0,
                                  mode="promise_in_bounds")
        w11 = jnp.take_along_axis(w11v, lsplat, axis=0,
                                  mode="promise_in_bounds")
        p = base + l
        for cg in range(C // L):
          sl = pl.ds(cg * L, L)
          obuf[obi, p, sl] = (
              w00 * vbuf[bufi, 0, p, sl] + w01 * vbuf[bufi, 1, p, sl]
              + w10 * vbuf[bufi, 2, p, sl] + w11 * vbuf[bufi, 3, p, sl])
        return 0

      lax.fori_loop(0, L, lane_body, 0, unroll=4)
      return 0

    lax.fori_loop(0, NG, group_body, 0)

  def out_slice(k, t):
    gbase = pl.multiple_of(tb + (i_base + k) * W + t * CH, CH)
    return out_hbm.at[pl.ds(gbase, CH)]

  def fire_write(k, t, obi):
    pltpu.async_copy(obuf.at[obi], out_slice(k, t), osem)

  def wait_write(k, t, obi):
    pltpu.make_async_copy(obuf.at[obi], out_slice(k, t), osem).wait()

  zero = jnp.int32(0)
  one = jnp.int32(1)
  pltpu.sync_copy(trf_hbm.at[pl.ds(flow_off(zero, 0), W)], flow_v.at[0, 0])
  pltpu.sync_copy(trf_hbm.at[pl.ds(flow_off(zero, 1), W)], flow_v.at[0, 1])
  compute_chunk(zero, zero, zero)
  fire_gathers(zero)
  fire_flow_prefetch(one)
  compute_chunk(zero, one, one)
  fire_gathers(one)

  def q_body(q, _):
    buf = lax.rem(q, NB)
    nq2 = q + 2
    nk2 = nq2 // NCHUNK
    nt2 = lax.rem(nq2, NCHUNK)
    k = q // NCHUNK
    t = lax.rem(q, NCHUNK)

    @pl.when(nq2 < NQ)
    def _stage_next():
      @pl.when(nt2 == 0)
      def _flow_ready():
        wait_flow_prefetch(nk2)

      nbuf = lax.rem(nq2, NB)
      compute_chunk(nk2, nt2, nbuf)
      fire_gathers(nbuf)

      @pl.when(jnp.logical_and(nt2 == 0, nk2 + 1 < ROWS_PER_W))
      def _flow_next():
        fire_flow_prefetch(nk2 + 1)

    wait_gathers(buf)
    combine(buf, lax.rem(q, 2))

    @pl.when(q >= 1)
    def _drain_prev_write():
      wait_write((q - 1) // NCHUNK, lax.rem(q - 1, NCHUNK),
                 lax.rem(q - 1, 2))

    fire_write(k, t, lax.rem(q, 2))
    return 0

  lax.fori_loop(0, NQ, q_body, 0)
  wait_write(jnp.int32(ROWS_PER_W - 1), jnp.int32(NCHUNK - 1),
             jnp.int32((NQ - 1) % 2))


@jax.jit
def kernel(vol, trf):
  table = jnp.transpose(vol, (0, 2, 3, 1)).reshape(B * HW, C)
  trf_flat = trf.reshape(B * 2 * H * W)

  mesh = plsc.VectorSubcoreMesh(core_axis_name="c", subcore_axis_name="s",
                                num_cores=NC, num_subcores=NS)
  f = pl.kernel(
      _sc_warp_kernel,
      out_type=jax.ShapeDtypeStruct((B * HW, C), jnp.float32),
      mesh=mesh,
      scratch_types=[
          pltpu.VMEM((2, 2, W), jnp.float32),
          pltpu.VMEM((NB, 4, CH), jnp.int32),
          pltpu.VMEM((NB, 4, CH), jnp.float32),
          pltpu.VMEM((NB, 4, CH, C), jnp.float32),
          pltpu.VMEM((2, CH, C), jnp.float32),
          pltpu.SemaphoreType.DMA,
          pltpu.SemaphoreType.DMA,
          pltpu.SemaphoreType.DMA,
      ],
      compiler_params=pltpu.CompilerParams(use_tc_tiling_on_sc=False),
  )
  out = f(table, trf_flat)
  return out.reshape(B, H, W, C)


# speedup vs baseline: 1.7298x; 1.1412x over previous
import jax
import jax.numpy as jnp
from jax import lax
from jax.experimental import pallas as pl
from jax.experimental.pallas import tpu as pltpu
from jax.experimental.pallas import tpu_sc as plsc

B, C, H, W = 4, 96, 96 * 4, 96 * 4
HW = H * W
NC, NS, L = 2, 16, 16
NW = NC * NS
ROWS_PER_W = (B * H) // NW
CH = 64
NCHUNK = W // CH
NQ = ROWS_PER_W * NCHUNK
NG = CH // L
NB = 3
CW = C // 2


def _sc_warp_kernel(table_hbm, trf_hbm, out_hbm,
                    flow_v, ibuf, wbuf, vbuf, obuf, gsem, osem, fsem):
  wid = lax.axis_index("s") * NC + lax.axis_index("c")
  b = wid // (NW // B)
  i_base = lax.rem(wid, NW // B) * ROWS_PER_W
  tb = b * HW

  iota = lax.iota(jnp.int32, L)
  iota_f = iota.astype(jnp.float32)

  def flow_off(k, z):
    return pl.multiple_of((b * 2 * H + z * H + (i_base + k)) * W, W)

  def fire_flow_prefetch(k):
    par = lax.rem(k, 2)
    pltpu.async_copy(trf_hbm.at[pl.ds(flow_off(k, 0), W)],
                     flow_v.at[par, 0], fsem)
    pltpu.async_copy(trf_hbm.at[pl.ds(flow_off(k, 1), W)],
                     flow_v.at[par, 1], fsem)

  def wait_flow_prefetch(k):
    par = lax.rem(k, 2)
    pltpu.make_async_copy(trf_hbm.at[pl.ds(flow_off(k, 0), W)],
                          flow_v.at[par, 0], fsem).wait()
    pltpu.make_async_copy(trf_hbm.at[pl.ds(flow_off(k, 1), W)],
                          flow_v.at[par, 1], fsem).wait()

  def compute_chunk(k, t, bufi):
    par = lax.rem(k, 2)
    i_f = (i_base + k).astype(jnp.float32)
    for g in range(NG):
      sl = pl.ds(g * L, L)
      jpos = t * CH + g * L
      fi = flow_v[par, 0, pl.ds(jpos, L)]
      fj = flow_v[par, 1, pl.ds(jpos, L)]
      loc_i = jnp.clip(i_f + fi, 0.0, float(H - 1))
      loc_j = jnp.clip(jpos.astype(jnp.float32) + iota_f + fj,
                       0.0, float(W - 1))
      i0 = jnp.minimum(loc_i.astype(jnp.int32), H - 2)
      j0 = jnp.minimum(loc_j.astype(jnp.int32), W - 2)
      wi = loc_i - i0.astype(jnp.float32)
      wj = loc_j - j0.astype(jnp.float32)
      base_idx = tb + i0 * W + j0
      ibuf[bufi, 0, sl] = base_idx
      ibuf[bufi, 1, sl] = base_idx + 1
      ibuf[bufi, 2, sl] = base_idx + W
      ibuf[bufi, 3, sl] = base_idx + (W + 1)
      wbuf[bufi, 0, sl] = (1.0 - wi) * (1.0 - wj)
      wbuf[bufi, 1, sl] = (1.0 - wi) * wj
      wbuf[bufi, 2, sl] = wi * (1.0 - wj)
      wbuf[bufi, 3, sl] = wi * wj

  def fire_gathers(bufi):
    for n in range(4):
      pltpu.async_copy(table_hbm.at[ibuf.at[bufi, n]], vbuf.at[bufi, n], gsem)

  def wait_gathers(bufi):
    for n in range(4):
      pltpu.make_async_copy(table_hbm.at[ibuf.at[bufi, n]],
                            vbuf.at[bufi, n], gsem).wait()

  def combine(bufi, obi):
    def group_body(g, _):
      base = g * L
      w00v = wbuf[bufi, 0, pl.ds(base, L)]
      w01v = wbuf[bufi, 1, pl.ds(base, L)]
      w10v = wbuf[bufi, 2, pl.ds(base, L)]
      w11v = wbuf[bufi, 3, pl.ds(base, L)]

      def lane_body(l, _):
        lsplat = jnp.full((L,), l, jnp.int32)
        w00 = jnp.take_along_axis(w00v, lsplat, axis=0,
                                  mode="promise_in_bounds")
        w01 = jnp.take_along_axis(w01v, lsplat, axis=0,
                                  mode="promise_in_bounds")
        w10 = jnp.take_along_axis(w10v, lsplat, axis=0,
                                  mode="promise_in_bounds")
        w11 = jnp.take_along_axis(w11v, lsplat, axis=0,
                                  mode="promise_in_bounds")
        p = base + l
        for cg in range(CW // L):
          sl = pl.ds(cg * L, L)
          acc_lo = jnp.zeros((L,), jnp.float32)
          acc_hi = jnp.zeros((L,), jnp.float32)
          for n, wn in ((0, w00), (1, w01), (2, w10), (3, w11)):
            wv = vbuf[bufi, n, p, sl]
            lo = lax.bitcast_convert_type(wv << 16, jnp.float32)
            hi = lax.bitcast_convert_type(
                wv & jnp.uint32(0xFFFF0000), jnp.float32)
            acc_lo = acc_lo + wn * lo
            acc_hi = acc_hi + wn * hi
          obuf[obi, p, sl] = acc_lo
          obuf[obi, p, pl.ds(CW + cg * L, L)] = acc_hi
        return 0

      lax.fori_loop(0, L, lane_body, 0, unroll=4)
      return 0

    lax.fori_loop(0, NG, group_body, 0)

  def out_slice(k, t):
    gbase = pl.multiple_of(tb + (i_base + k) * W + t * CH, CH)
    return out_hbm.at[pl.ds(gbase, CH)]

  def fire_write(k, t, obi):
    pltpu.async_copy(obuf.at[obi], out_slice(k, t), osem)

  def wait_write(k, t, obi):
    pltpu.make_async_copy(obuf.at[obi], out_slice(k, t), osem).wait()

  zero = jnp.int32(0)
  one = jnp.int32(1)
  pltpu.sync_copy(trf_hbm.at[pl.ds(flow_off(zero, 0), W)], flow_v.at[0, 0])
  pltpu.sync_copy(trf_hbm.at[pl.ds(flow_off(zero, 1), W)], flow_v.at[0, 1])
  compute_chunk(zero, zero, zero)
  fire_gathers(zero)
  fire_flow_prefetch(one)
  compute_chunk(zero, one, one)
  fire_gathers(one)

  def q_body(q, _):
    buf = lax.rem(q, NB)
    nq2 = q + 2
    nk2 = nq2 // NCHUNK
    nt2 = lax.rem(nq2, NCHUNK)
    k = q // NCHUNK
    t = lax.rem(q, NCHUNK)

    @pl.when(nq2 < NQ)
    def _stage_next():
      @pl.when(nt2 == 0)
      def _flow_ready():
        wait_flow_prefetch(nk2)

      nbuf = lax.rem(nq2, NB)
      compute_chunk(nk2, nt2, nbuf)
      fire_gathers(nbuf)

      @pl.when(jnp.logical_and(nt2 == 0, nk2 + 1 < ROWS_PER_W))
      def _flow_next():
        fire_flow_prefetch(nk2 + 1)

    wait_gathers(buf)
    combine(buf, lax.rem(q, 2))

    @pl.when(q >= 1)
    def _drain_prev_write():
      wait_write((q - 1) // NCHUNK, lax.rem(q - 1, NCHUNK),
                 lax.rem(q - 1, 2))

    fire_write(k, t, lax.rem(q, 2))
    return 0

  lax.fori_loop(0, NQ, q_body, 0)
  wait_write(jnp.int32(ROWS_PER_W - 1), jnp.int32(NCHUNK - 1),
             jnp.int32((NQ - 1) % 2))


@jax.jit
def kernel(vol, trf):
  vol_t = jnp.transpose(vol, (0, 2, 3, 1)).reshape(B * HW, C)
  lo = lax.bitcast_convert_type(
      vol_t[:, :CW].astype(jnp.bfloat16), jnp.uint16).astype(jnp.uint32)
  hi = lax.bitcast_convert_type(
      vol_t[:, CW:].astype(jnp.bfloat16), jnp.uint16).astype(jnp.uint32)
  table = lo | (hi << 16)
  trf_flat = trf.reshape(B * 2 * H * W)

  mesh = plsc.VectorSubcoreMesh(core_axis_name="c", subcore_axis_name="s",
                                num_cores=NC, num_subcores=NS)
  f = pl.kernel(
      _sc_warp_kernel,
      out_type=jax.ShapeDtypeStruct((B * HW, C), jnp.float32),
      mesh=mesh,
      scratch_types=[
          pltpu.VMEM((2, 2, W), jnp.float32),
          pltpu.VMEM((NB, 4, CH), jnp.int32),
          pltpu.VMEM((NB, 4, CH), jnp.float32),
          pltpu.VMEM((NB, 4, CH, CW), jnp.uint32),
          pltpu.VMEM((2, CH, C), jnp.float32),
          pltpu.SemaphoreType.DMA,
          pltpu.SemaphoreType.DMA,
          pltpu.SemaphoreType.DMA,
      ],
      compiler_params=pltpu.CompilerParams(use_tc_tiling_on_sc=False),
  )
  out = f(table, trf_flat)
  return out.reshape(B, H, W, C)


# speedup vs baseline: 1.8305x; 1.0582x over previous
import jax
import jax.numpy as jnp
from jax import lax
from jax.experimental import pallas as pl
from jax.experimental.pallas import tpu as pltpu
from jax.experimental.pallas import tpu_sc as plsc

B, C, H, W = 4, 96, 96 * 4, 96 * 4
HW = H * W
NC, NS, L = 2, 16, 16
NW = NC * NS
ROWS_PER_W = (B * H) // NW
CH = 64
NCHUNK = W // CH
NQ = ROWS_PER_W * NCHUNK
NG = CH // L
NB = 3
CW = C // 2


def _sc_warp_kernel(table_hbm, trf_hbm, out_hbm,
                    flow_v, ibuf, wbuf, vbuf, obuf, gsem, osem, fsem):
  wid = lax.axis_index("s") * NC + lax.axis_index("c")
  b = wid // (NW // B)
  i_base = lax.rem(wid, NW // B) * ROWS_PER_W
  tb = b * HW

  iota = lax.iota(jnp.int32, L)
  iota_f = iota.astype(jnp.float32)

  def flow_off(k, z):
    return pl.multiple_of((b * 2 * H + z * H + (i_base + k)) * W, W)

  def fire_flow_prefetch(k):
    par = lax.rem(k, 2)
    pltpu.async_copy(trf_hbm.at[pl.ds(flow_off(k, 0), W)],
                     flow_v.at[par, 0], fsem)
    pltpu.async_copy(trf_hbm.at[pl.ds(flow_off(k, 1), W)],
                     flow_v.at[par, 1], fsem)

  def wait_flow_prefetch(k):
    par = lax.rem(k, 2)
    pltpu.make_async_copy(trf_hbm.at[pl.ds(flow_off(k, 0), W)],
                          flow_v.at[par, 0], fsem).wait()
    pltpu.make_async_copy(trf_hbm.at[pl.ds(flow_off(k, 1), W)],
                          flow_v.at[par, 1], fsem).wait()

  def compute_chunk(k, t, bufi):
    par = lax.rem(k, 2)
    i_f = (i_base + k).astype(jnp.float32)
    for g in range(NG):
      sl = pl.ds(g * L, L)
      jpos = t * CH + g * L
      fi = flow_v[par, 0, pl.ds(jpos, L)]
      fj = flow_v[par, 1, pl.ds(jpos, L)]
      loc_i = jnp.clip(i_f + fi, 0.0, float(H - 1))
      loc_j = jnp.clip(jpos.astype(jnp.float32) + iota_f + fj,
                       0.0, float(W - 1))
      i0 = jnp.minimum(loc_i.astype(jnp.int32), H - 2)
      j0 = jnp.minimum(loc_j.astype(jnp.int32), W - 2)
      wi = loc_i - i0.astype(jnp.float32)
      wj = loc_j - j0.astype(jnp.float32)
      base_idx = tb + i0 * W + j0
      ibuf[bufi, 0, sl] = base_idx
      ibuf[bufi, 1, sl] = base_idx + 1
      ibuf[bufi, 2, sl] = base_idx + W
      ibuf[bufi, 3, sl] = base_idx + (W + 1)
      wbuf[bufi, 0, sl] = (1.0 - wi) * (1.0 - wj)
      wbuf[bufi, 1, sl] = (1.0 - wi) * wj
      wbuf[bufi, 2, sl] = wi * (1.0 - wj)
      wbuf[bufi, 3, sl] = wi * wj

  def fire_gathers(bufi):
    for n in range(4):
      pltpu.async_copy(table_hbm.at[ibuf.at[bufi, n]], vbuf.at[bufi, n], gsem)

  def wait_gathers(bufi):
    for n in range(4):
      pltpu.make_async_copy(table_hbm.at[ibuf.at[bufi, n]],
                            vbuf.at[bufi, n], gsem).wait()

  def combine(bufi, obi):
    def group_body(g, _):
      base = g * L
      w00v = wbuf[bufi, 0, pl.ds(base, L)]
      w01v = wbuf[bufi, 1, pl.ds(base, L)]
      w10v = wbuf[bufi, 2, pl.ds(base, L)]
      w11v = wbuf[bufi, 3, pl.ds(base, L)]

      def lane_body(l, _):
        lsplat = jnp.full((L,), l, jnp.int32)
        w00 = jnp.take_along_axis(w00v, lsplat, axis=0,
                                  mode="promise_in_bounds")
        w01 = jnp.take_along_axis(w01v, lsplat, axis=0,
                                  mode="promise_in_bounds")
        w10 = jnp.take_along_axis(w10v, lsplat, axis=0,
                                  mode="promise_in_bounds")
        w11 = jnp.take_along_axis(w11v, lsplat, axis=0,
                                  mode="promise_in_bounds")
        p = base + l
        for cg in range(CW // L):
          sl = pl.ds(cg * L, L)
          acc_lo = jnp.zeros((L,), jnp.float32)
          acc_hi = jnp.zeros((L,), jnp.float32)
          for n, wn in ((0, w00), (1, w01), (2, w10), (3, w11)):
            wv = vbuf[bufi, n, p, sl]
            lo = lax.bitcast_convert_type(wv << 16, jnp.float32)
            hi = lax.bitcast_convert_type(
                wv & jnp.uint32(0xFFFF0000), jnp.float32)
            acc_lo = acc_lo + wn * lo
            acc_hi = acc_hi + wn * hi
          obuf[obi, p, sl] = acc_lo
          obuf[obi, p, pl.ds(CW + cg * L, L)] = acc_hi
        return 0

      lax.fori_loop(0, L, lane_body, 0, unroll=4)
      return 0

    lax.fori_loop(0, NG, group_body, 0)

  def out_slice(k, t):
    gbase = pl.multiple_of(tb + (i_base + k) * W + t * CH, CH)
    return out_hbm.at[pl.ds(gbase, CH)]

  def fire_write(k, t, obi):
    pltpu.async_copy(obuf.at[obi], out_slice(k, t), osem)

  def wait_write(k, t, obi):
    pltpu.make_async_copy(obuf.at[obi], out_slice(k, t), osem).wait()

  zero = jnp.int32(0)
  one = jnp.int32(1)
  pltpu.sync_copy(trf_hbm.at[pl.ds(flow_off(zero, 0), W)], flow_v.at[0, 0])
  pltpu.sync_copy(trf_hbm.at[pl.ds(flow_off(zero, 1), W)], flow_v.at[0, 1])
  compute_chunk(zero, zero, zero)
  fire_gathers(zero)
  fire_flow_prefetch(one)
  compute_chunk(zero, one, one)
  fire_gathers(one)

  def q_body(q, _):
    buf = lax.rem(q, NB)
    nq2 = q + 2
    nk2 = nq2 // NCHUNK
    nt2 = lax.rem(nq2, NCHUNK)
    k = q // NCHUNK
    t = lax.rem(q, NCHUNK)

    @pl.when(nq2 < NQ)
    def _stage_next():
      @pl.when(nt2 == 0)
      def _flow_ready():
        wait_flow_prefetch(nk2)

      nbuf = lax.rem(nq2, NB)
      compute_chunk(nk2, nt2, nbuf)
      fire_gathers(nbuf)

      @pl.when(jnp.logical_and(nt2 == 0, nk2 + 1 < ROWS_PER_W))
      def _flow_next():
        fire_flow_prefetch(nk2 + 1)

    wait_gathers(buf)
    combine(buf, lax.rem(q, 2))

    @pl.when(q >= 1)
    def _drain_prev_write():
      wait_write((q - 1) // NCHUNK, lax.rem(q - 1, NCHUNK),
                 lax.rem(q - 1, 2))

    fire_write(k, t, lax.rem(q, 2))
    return 0

  lax.fori_loop(0, NQ, q_body, 0)
  wait_write(jnp.int32(ROWS_PER_W - 1), jnp.int32(NCHUNK - 1),
             jnp.int32((NQ - 1) % 2))


@jax.jit
def kernel(vol, trf):
  lo = lax.bitcast_convert_type(
      vol[:, :CW].astype(jnp.bfloat16), jnp.uint16).astype(jnp.uint32)
  hi = lax.bitcast_convert_type(
      vol[:, CW:].astype(jnp.bfloat16), jnp.uint16).astype(jnp.uint32)
  packed = lo | (hi << 16)
  table = jnp.transpose(packed, (0, 2, 3, 1)).reshape(B * HW, CW)
  trf_flat = trf.reshape(B * 2 * H * W)

  mesh = plsc.VectorSubcoreMesh(core_axis_name="c", subcore_axis_name="s",
                                num_cores=NC, num_subcores=NS)
  f = pl.kernel(
      _sc_warp_kernel,
      out_type=jax.ShapeDtypeStruct((B * HW, C), jnp.float32),
      mesh=mesh,
      scratch_types=[
          pltpu.VMEM((2, 2, W), jnp.float32),
          pltpu.VMEM((NB, 4, CH), jnp.int32),
          pltpu.VMEM((NB, 4, CH), jnp.float32),
          pltpu.VMEM((NB, 4, CH, CW), jnp.uint32),
          pltpu.VMEM((2, CH, C), jnp.float32),
          pltpu.SemaphoreType.DMA,
          pltpu.SemaphoreType.DMA,
          pltpu.SemaphoreType.DMA,
      ],
      compiler_params=pltpu.CompilerParams(use_tc_tiling_on_sc=False),
  )
  out = f(table, trf_flat)
  return out.reshape(B, H, W, C)


# speedup vs baseline: 1.8323x; 1.0010x over previous
import jax
import jax.numpy as jnp
from jax import lax
from jax.experimental import pallas as pl
from jax.experimental.pallas import tpu as pltpu
from jax.experimental.pallas import tpu_sc as plsc

B, C, H, W = 4, 96, 96 * 4, 96 * 4
HW = H * W
NC, NS, L = 2, 16, 16
NW = NC * NS
ROWS_PER_W = (B * H) // NW
CH = 96
NCHUNK = W // CH
NQ = ROWS_PER_W * NCHUNK
NG = CH // L
NB = 3
CW = C // 2


def _sc_warp_kernel(table_hbm, trf_hbm, out_hbm,
                    flow_v, ibuf, wbuf, vbuf, obuf, gsem, osem, fsem):
  wid = lax.axis_index("s") * NC + lax.axis_index("c")
  b = wid // (NW // B)
  i_base = lax.rem(wid, NW // B) * ROWS_PER_W
  tb = b * HW

  iota = lax.iota(jnp.int32, L)
  iota_f = iota.astype(jnp.float32)

  def flow_off(k, z):
    return pl.multiple_of((b * 2 * H + z * H + (i_base + k)) * W, W)

  def fire_flow_prefetch(k):
    par = lax.rem(k, 2)
    pltpu.async_copy(trf_hbm.at[pl.ds(flow_off(k, 0), W)],
                     flow_v.at[par, 0], fsem)
    pltpu.async_copy(trf_hbm.at[pl.ds(flow_off(k, 1), W)],
                     flow_v.at[par, 1], fsem)

  def wait_flow_prefetch(k):
    par = lax.rem(k, 2)
    pltpu.make_async_copy(trf_hbm.at[pl.ds(flow_off(k, 0), W)],
                          flow_v.at[par, 0], fsem).wait()
    pltpu.make_async_copy(trf_hbm.at[pl.ds(flow_off(k, 1), W)],
                          flow_v.at[par, 1], fsem).wait()

  def compute_chunk(k, t, bufi):
    par = lax.rem(k, 2)
    i_f = (i_base + k).astype(jnp.float32)
    for g in range(NG):
      sl = pl.ds(g * L, L)
      jpos = t * CH + g * L
      fi = flow_v[par, 0, pl.ds(jpos, L)]
      fj = flow_v[par, 1, pl.ds(jpos, L)]
      loc_i = jnp.clip(i_f + fi, 0.0, float(H - 1))
      loc_j = jnp.clip(jpos.astype(jnp.float32) + iota_f + fj,
                       0.0, float(W - 1))
      i0 = jnp.minimum(loc_i.astype(jnp.int32), H - 2)
      j0 = jnp.minimum(loc_j.astype(jnp.int32), W - 2)
      wi = loc_i - i0.astype(jnp.float32)
      wj = loc_j - j0.astype(jnp.float32)
      base_idx = tb + i0 * W + j0
      ibuf[bufi, 0, sl] = base_idx
      ibuf[bufi, 1, sl] = base_idx + 1
      ibuf[bufi, 2, sl] = base_idx + W
      ibuf[bufi, 3, sl] = base_idx + (W + 1)
      wbuf[bufi, 0, sl] = (1.0 - wi) * (1.0 - wj)
      wbuf[bufi, 1, sl] = (1.0 - wi) * wj
      wbuf[bufi, 2, sl] = wi * (1.0 - wj)
      wbuf[bufi, 3, sl] = wi * wj

  def fire_gathers(bufi):
    for n in range(4):
      pltpu.async_copy(table_hbm.at[ibuf.at[bufi, n]], vbuf.at[bufi, n], gsem)

  def wait_gathers(bufi):
    for n in range(4):
      pltpu.make_async_copy(table_hbm.at[ibuf.at[bufi, n]],
                            vbuf.at[bufi, n], gsem).wait()

  def combine(bufi, obi):
    def group_body(g, _):
      base = g * L
      w00v = wbuf[bufi, 0, pl.ds(base, L)]
      w01v = wbuf[bufi, 1, pl.ds(base, L)]
      w10v = wbuf[bufi, 2, pl.ds(base, L)]
      w11v = wbuf[bufi, 3, pl.ds(base, L)]

      def lane_body(l, _):
        lsplat = jnp.full((L,), l, jnp.int32)
        w00 = jnp.take_along_axis(w00v, lsplat, axis=0,
                                  mode="promise_in_bounds")
        w01 = jnp.take_along_axis(w01v, lsplat, axis=0,
                                  mode="promise_in_bounds")
        w10 = jnp.take_along_axis(w10v, lsplat, axis=0,
                                  mode="promise_in_bounds")
        w11 = jnp.take_along_axis(w11v, lsplat, axis=0,
                                  mode="promise_in_bounds")
        p = base + l
        for cg in range(CW // L):
          sl = pl.ds(cg * L, L)
          acc_lo = jnp.zeros((L,), jnp.float32)
          acc_hi = jnp.zeros((L,), jnp.float32)
          for n, wn in ((0, w00), (1, w01), (2, w10), (3, w11)):
            wv = vbuf[bufi, n, p, sl]
            lo = lax.bitcast_convert_type(wv << 16, jnp.float32)
            hi = lax.bitcast_convert_type(
                wv & jnp.uint32(0xFFFF0000), jnp.float32)
            acc_lo = acc_lo + wn * lo
            acc_hi = acc_hi + wn * hi
          obuf[obi, p, sl] = acc_lo
          obuf[obi, p, pl.ds(CW + cg * L, L)] = acc_hi
        return 0

      lax.fori_loop(0, L, lane_body, 0, unroll=4)
      return 0

    lax.fori_loop(0, NG, group_body, 0)

  def out_slice(k, t):
    gbase = pl.multiple_of(tb + (i_base + k) * W + t * CH, CH)
    return out_hbm.at[pl.ds(gbase, CH)]

  def fire_write(k, t, obi):
    pltpu.async_copy(obuf.at[obi], out_slice(k, t), osem)

  def wait_write(k, t, obi):
    pltpu.make_async_copy(obuf.at[obi], out_slice(k, t), osem).wait()

  zero = jnp.int32(0)
  one = jnp.int32(1)
  pltpu.sync_copy(trf_hbm.at[pl.ds(flow_off(zero, 0), W)], flow_v.at[0, 0])
  pltpu.sync_copy(trf_hbm.at[pl.ds(flow_off(zero, 1), W)], flow_v.at[0, 1])
  compute_chunk(zero, zero, zero)
  fire_gathers(zero)
  fire_flow_prefetch(one)
  compute_chunk(zero, one, one)
  fire_gathers(one)

  def q_body(q, _):
    buf = lax.rem(q, NB)
    nq2 = q + 2
    nk2 = nq2 // NCHUNK
    nt2 = lax.rem(nq2, NCHUNK)
    k = q // NCHUNK
    t = lax.rem(q, NCHUNK)

    @pl.when(nq2 < NQ)
    def _stage_next():
      @pl.when(nt2 == 0)
      def _flow_ready():
        wait_flow_prefetch(nk2)

      nbuf = lax.rem(nq2, NB)
      compute_chunk(nk2, nt2, nbuf)
      fire_gathers(nbuf)

      @pl.when(jnp.logical_and(nt2 == 0, nk2 + 1 < ROWS_PER_W))
      def _flow_next():
        fire_flow_prefetch(nk2 + 1)

    wait_gathers(buf)
    combine(buf, lax.rem(q, 2))

    @pl.when(q >= 1)
    def _drain_prev_write():
      wait_write((q - 1) // NCHUNK, lax.rem(q - 1, NCHUNK),
                 lax.rem(q - 1, 2))

    fire_write(k, t, lax.rem(q, 2))
    return 0

  lax.fori_loop(0, NQ, q_body, 0)
  wait_write(jnp.int32(ROWS_PER_W - 1), jnp.int32(NCHUNK - 1),
             jnp.int32((NQ - 1) % 2))


@jax.jit
def kernel(vol, trf):
  lo = lax.bitcast_convert_type(
      vol[:, :CW].astype(jnp.bfloat16), jnp.uint16).astype(jnp.uint32)
  hi = lax.bitcast_convert_type(
      vol[:, CW:].astype(jnp.bfloat16), jnp.uint16).astype(jnp.uint32)
  packed = lo | (hi << 16)
  table = jnp.transpose(packed, (0, 2, 3, 1)).reshape(B * HW, CW)
  trf_flat = trf.reshape(B * 2 * H * W)

  mesh = plsc.VectorSubcoreMesh(core_axis_name="c", subcore_axis_name="s",
                                num_cores=NC, num_subcores=NS)
  f = pl.kernel(
      _sc_warp_kernel,
      out_type=jax.ShapeDtypeStruct((B * HW, C), jnp.float32),
      mesh=mesh,
      scratch_types=[
          pltpu.VMEM((2, 2, W), jnp.float32),
          pltpu.VMEM((NB, 4, CH), jnp.int32),
          pltpu.VMEM((NB, 4, CH), jnp.float32),
          pltpu.VMEM((NB, 4, CH, CW), jnp.uint32),
          pltpu.VMEM((2, CH, C), jnp.float32),
          pltpu.SemaphoreType.DMA,
          pltpu.SemaphoreType.DMA,
          pltpu.SemaphoreType.DMA,
      ],
      compiler_params=pltpu.CompilerParams(use_tc_tiling_on_sc=False),
  )
  out = f(table, trf_flat)
  return out.reshape(B, H, W, C)


# speedup vs baseline: 1.8341x; 1.0010x over previous
import jax
import jax.numpy as jnp
from jax import lax
from jax.experimental import pallas as pl
from jax.experimental.pallas import tpu as pltpu
from jax.experimental.pallas import tpu_sc as plsc

B, C, H, W = 4, 96, 96 * 4, 96 * 4
HW = H * W
NC, NS, L = 2, 16, 16
NW = NC * NS
ROWS_PER_W = (B * H) // NW
CH = 96
NCHUNK = W // CH
NQ = ROWS_PER_W * NCHUNK
NG = CH // L
NB = 4
CW = C // 2


def _sc_warp_kernel(table_hbm, trf_hbm, out_hbm,
                    flow_v, ibuf, wbuf, vbuf, obuf, gsem, osem, fsem):
  wid = lax.axis_index("s") * NC + lax.axis_index("c")
  b = wid // (NW // B)
  i_base = lax.rem(wid, NW // B) * ROWS_PER_W
  tb = b * HW

  iota = lax.iota(jnp.int32, L)
  iota_f = iota.astype(jnp.float32)

  def flow_off(k, z):
    return pl.multiple_of((b * 2 * H + z * H + (i_base + k)) * W, W)

  def fire_flow_prefetch(k):
    par = lax.rem(k, 2)
    pltpu.async_copy(trf_hbm.at[pl.ds(flow_off(k, 0), W)],
                     flow_v.at[par, 0], fsem)
    pltpu.async_copy(trf_hbm.at[pl.ds(flow_off(k, 1), W)],
                     flow_v.at[par, 1], fsem)

  def wait_flow_prefetch(k):
    par = lax.rem(k, 2)
    pltpu.make_async_copy(trf_hbm.at[pl.ds(flow_off(k, 0), W)],
                          flow_v.at[par, 0], fsem).wait()
    pltpu.make_async_copy(trf_hbm.at[pl.ds(flow_off(k, 1), W)],
                          flow_v.at[par, 1], fsem).wait()

  def compute_chunk(k, t, bufi):
    par = lax.rem(k, 2)
    i_f = (i_base + k).astype(jnp.float32)
    for g in range(NG):
      sl = pl.ds(g * L, L)
      jpos = t * CH + g * L
      fi = flow_v[par, 0, pl.ds(jpos, L)]
      fj = flow_v[par, 1, pl.ds(jpos, L)]
      loc_i = jnp.clip(i_f + fi, 0.0, float(H - 1))
      loc_j = jnp.clip(jpos.astype(jnp.float32) + iota_f + fj,
                       0.0, float(W - 1))
      i0 = jnp.minimum(loc_i.astype(jnp.int32), H - 2)
      j0 = jnp.minimum(loc_j.astype(jnp.int32), W - 2)
      wi = loc_i - i0.astype(jnp.float32)
      wj = loc_j - j0.astype(jnp.float32)
      base_idx = tb + i0 * W + j0
      ibuf[bufi, 0, sl] = base_idx
      ibuf[bufi, 1, sl] = base_idx + 1
      ibuf[bufi, 2, sl] = base_idx + W
      ibuf[bufi, 3, sl] = base_idx + (W + 1)
      wbuf[bufi, 0, sl] = (1.0 - wi) * (1.0 - wj)
      wbuf[bufi, 1, sl] = (1.0 - wi) * wj
      wbuf[bufi, 2, sl] = wi * (1.0 - wj)
      wbuf[bufi, 3, sl] = wi * wj

  def fire_gathers(bufi):
    for n in range(4):
      pltpu.async_copy(table_hbm.at[ibuf.at[bufi, n]], vbuf.at[bufi, n], gsem)

  def wait_gathers(bufi):
    for n in range(4):
      pltpu.make_async_copy(table_hbm.at[ibuf.at[bufi, n]],
                            vbuf.at[bufi, n], gsem).wait()

  def combine(bufi, obi):
    def group_body(g, _):
      base = g * L
      w00v = wbuf[bufi, 0, pl.ds(base, L)]
      w01v = wbuf[bufi, 1, pl.ds(base, L)]
      w10v = wbuf[bufi, 2, pl.ds(base, L)]
      w11v = wbuf[bufi, 3, pl.ds(base, L)]

      def lane_body(l, _):
        lsplat = jnp.full((L,), l, jnp.int32)
        w00 = jnp.take_along_axis(w00v, lsplat, axis=0,
                                  mode="promise_in_bounds")
        w01 = jnp.take_along_axis(w01v, lsplat, axis=0,
                                  mode="promise_in_bounds")
        w10 = jnp.take_along_axis(w10v, lsplat, axis=0,
                                  mode="promise_in_bounds")
        w11 = jnp.take_along_axis(w11v, lsplat, axis=0,
                                  mode="promise_in_bounds")
        p = base + l
        for cg in range(CW // L):
          sl = pl.ds(cg * L, L)
          acc_lo = jnp.zeros((L,), jnp.float32)
          acc_hi = jnp.zeros((L,), jnp.float32)
          for n, wn in ((0, w00), (1, w01), (2, w10), (3, w11)):
            wv = vbuf[bufi, n, p, sl]
            lo = lax.bitcast_convert_type(wv << 16, jnp.float32)
            hi = lax.bitcast_convert_type(
                wv & jnp.uint32(0xFFFF0000), jnp.float32)
            acc_lo = acc_lo + wn * lo
            acc_hi = acc_hi + wn * hi
          obuf[obi, p, sl] = acc_lo
          obuf[obi, p, pl.ds(CW + cg * L, L)] = acc_hi
        return 0

      lax.fori_loop(0, L, lane_body, 0, unroll=4)
      return 0

    lax.fori_loop(0, NG, group_body, 0)

  def out_slice(k, t):
    gbase = pl.multiple_of(tb + (i_base + k) * W + t * CH, CH)
    return out_hbm.at[pl.ds(gbase, CH)]

  def fire_write(k, t, obi):
    pltpu.async_copy(obuf.at[obi], out_slice(k, t), osem)

  def wait_write(k, t, obi):
    pltpu.make_async_copy(obuf.at[obi], out_slice(k, t), osem).wait()

  zero = jnp.int32(0)
  one = jnp.int32(1)
  pltpu.sync_copy(trf_hbm.at[pl.ds(flow_off(zero, 0), W)], flow_v.at[0, 0])
  pltpu.sync_copy(trf_hbm.at[pl.ds(flow_off(zero, 1), W)], flow_v.at[0, 1])
  compute_chunk(zero, zero, zero)
  fire_gathers(zero)
  fire_flow_prefetch(one)
  compute_chunk(zero, one, one)
  fire_gathers(one)

  def q_body(q, _):
    buf = lax.rem(q, NB)
    nq2 = q + 2
    nk2 = nq2 // NCHUNK
    nt2 = lax.rem(nq2, NCHUNK)
    k = q // NCHUNK
    t = lax.rem(q, NCHUNK)

    @pl.when(nq2 < NQ)
    def _stage_next():
      @pl.when(nt2 == 0)
      def _flow_ready():
        wait_flow_prefetch(nk2)

      nbuf = lax.rem(nq2, NB)
      compute_chunk(nk2, nt2, nbuf)
      fire_gathers(nbuf)

      @pl.when(jnp.logical_and(nt2 == 0, nk2 + 1 < ROWS_PER_W))
      def _flow_next():
        fire_flow_prefetch(nk2 + 1)

    wait_gathers(buf)
    combine(buf, lax.rem(q, 2))

    @pl.when(q >= 1)
    def _drain_prev_write():
      wait_write((q - 1) // NCHUNK, lax.rem(q - 1, NCHUNK),
                 lax.rem(q - 1, 2))

    fire_write(k, t, lax.rem(q, 2))
    return 0

  lax.fori_loop(0, NQ, q_body, 0)
  wait_write(jnp.int32(ROWS_PER_W - 1), jnp.int32(NCHUNK - 1),
             jnp.int32((NQ - 1) % 2))


@jax.jit
def kernel(vol, trf):
  lo = lax.bitcast_convert_type(
      vol[:, :CW].astype(jnp.bfloat16), jnp.uint16).astype(jnp.uint32)
  hi = lax.bitcast_convert_type(
      vol[:, CW:].astype(jnp.bfloat16), jnp.uint16).astype(jnp.uint32)
  packed = lo | (hi << 16)
  table = jnp.transpose(packed, (0, 2, 3, 1)).reshape(B * HW, CW)
  trf_flat = trf.reshape(B * 2 * H * W)

  mesh = plsc.VectorSubcoreMesh(core_axis_name="c", subcore_axis_name="s",
                                num_cores=NC, num_subcores=NS)
  f = pl.kernel(
      _sc_warp_kernel,
      out_type=jax.ShapeDtypeStruct((B * HW, C), jnp.float32),
      mesh=mesh,
      scratch_types=[
          pltpu.VMEM((2, 2, W), jnp.float32),
          pltpu.VMEM((NB, 4, CH), jnp.int32),
          pltpu.VMEM((NB, 4, CH), jnp.float32),
          pltpu.VMEM((NB, 4, CH, CW), jnp.uint32),
          pltpu.VMEM((2, CH, C), jnp.float32),
          pltpu.SemaphoreType.DMA,
          pltpu.SemaphoreType.DMA,
          pltpu.SemaphoreType.DMA,
      ],
      compiler_params=pltpu.CompilerParams(use_tc_tiling_on_sc=False),
  )
  out = f(table, trf_flat)
  return out.reshape(B, H, W, C)
